# baseline (speedup 1.0000x reference)
import jax
import jax.numpy as jnp
from jax import lax
from jax.experimental import pallas as pl
from jax.experimental.pallas import tpu as pltpu
from jax.experimental.pallas import tpu_sc as plsc

_N = 10000
_E = 160000
_D = 256
_H = 128
_CHUNK = 100
_JCH = 100
_RPT = 640
_NPAD = 10240
_WB = 80


def _sc_agg_kernel(x_lo, x_hi, src3d, dst3d, zrow, zdeg, onesh,
                   agg_lo, agg_hi, deg_out,
                   idx_src, idx_dst, rows, ones_v, dbuf,
                   sga, sgb, ssa, ssb, sd,
                   acc_sh, deg_sh):
    c = lax.axis_index("c")
    s = lax.axis_index("s")

    pltpu.sync_copy(src3d.at[s], idx_src)
    pltpu.sync_copy(dst3d.at[s], idx_dst)

    def run(x_tab, agg_out, with_deg):
        pltpu.sync_copy(zrow, acc_sh.at[pl.ds(s * _RPT, _RPT)])
        if with_deg:
            pltpu.sync_copy(zdeg.at[pl.ds(s * _RPT, _RPT)],
                            deg_sh.at[pl.ds(s * _RPT, _RPT)])
            pltpu.sync_copy(onesh, ones_v)
        plsc.subcore_barrier()

        def gather(j, buf, sem):
            pltpu.async_copy(x_tab.at[idx_src.at[j]], rows.at[buf], sem)

        def gather_wait(j, buf, sem):
            pltpu.make_async_copy(x_tab.at[idx_src.at[j]], rows.at[buf],
                                  sem).wait()

        def scatter_start(j, buf, sem):
            ds = [pltpu.async_copy(rows.at[buf], acc_sh.at[idx_dst.at[j]],
                                   sem, add=True)]
            if with_deg:
                ds.append(pltpu.async_copy(ones_v, deg_sh.at[idx_dst.at[j]],
                                           sd, add=True))
            return ds

        gather(0, 0, sga)
        gather(1, 1, sgb)

        def body(i, carry):
            a = 2 * i
            gather_wait(a, 0, sga)
            da = scatter_start(a, 0, ssa)
            gather_wait(a + 1, 1, sgb)
            db = scatter_start(a + 1, 1, ssb)
            for d in da:
                d.wait()

            @pl.when(a + 2 < _JCH)
            def _():
                gather(a + 2, 0, sga)
            for d in db:
                d.wait()

            @pl.when(a + 3 < _JCH)
            def _():
                gather(a + 3, 1, sgb)
            return carry

        lax.fori_loop(0, _JCH // 2, body, 0)
        if _JCH % 2 == 1:
            gather_wait(_JCH - 1, 0, sga)
            for d in scatter_start(_JCH - 1, 0, ssa):
                d.wait()

        plsc.subcore_barrier()

        pltpu.sync_copy(acc_sh.at[pl.ds(s * _RPT, _WB)],
                        rows.at[0, pl.ds(0, _WB)])
        for k in range(8):
            b = k % 2
            if k < 7:
                r1 = s * _RPT + (k + 1) * _WB
                pltpu.async_copy(acc_sh.at[pl.ds(r1, _WB)],
                                 rows.at[1 - b, pl.ds(0, _WB)], sga)
            r0 = s * _RPT + k * _WB
            pltpu.sync_copy(rows.at[b, pl.ds(0, _WB)],
                            agg_out.at[pl.ds(r0, _WB)])
            if k < 7:
                r1 = s * _RPT + (k + 1) * _WB
                pltpu.make_async_copy(acc_sh.at[pl.ds(r1, _WB)],
                                      rows.at[1 - b, pl.ds(0, _WB)],
                                      sga).wait()
        if with_deg:
            pltpu.sync_copy(deg_sh.at[pl.ds(s * _RPT, _RPT)], dbuf)
            pltpu.sync_copy(dbuf, deg_out.at[pl.ds(s * _RPT, _RPT)])

    @pl.when(c == 0)
    def _():
        run(x_lo, agg_lo, True)

    @pl.when(c == 1)
    def _():
        run(x_hi, agg_hi, False)


_sc_agg = pl.kernel(
    _sc_agg_kernel,
    out_type=[
        jax.ShapeDtypeStruct((_NPAD, _H), jnp.float32),
        jax.ShapeDtypeStruct((_NPAD, _H), jnp.float32),
        jax.ShapeDtypeStruct((_NPAD,), jnp.float32),
    ],
    mesh=plsc.VectorSubcoreMesh(core_axis_name="c", subcore_axis_name="s"),
    compiler_params=pltpu.CompilerParams(use_tc_tiling_on_sc=False),
    scratch_types=[
        pltpu.VMEM((_JCH, _CHUNK), jnp.int32),
        pltpu.VMEM((_JCH, _CHUNK), jnp.int32),
        pltpu.VMEM((2, _CHUNK, _H), jnp.float32),
        pltpu.VMEM((_CHUNK,), jnp.float32),
        pltpu.VMEM((_RPT,), jnp.float32),
        pltpu.SemaphoreType.DMA,
        pltpu.SemaphoreType.DMA,
        pltpu.SemaphoreType.DMA,
        pltpu.SemaphoreType.DMA,
        pltpu.SemaphoreType.DMA,
        pltpu.VMEM_SHARED((_NPAD, _H), jnp.float32),
        pltpu.VMEM_SHARED((_NPAD,), jnp.float32),
    ],
)


_BLK = 1000


def _tc_root_kernel(x_ref, wr_ref, b_ref, o_ref):
    o_ref[...] = lax.dot_general(
        x_ref[...], wr_ref[...], (((1,), (1,)), ((), ())),
        preferred_element_type=jnp.float32) + b_ref[...]


def _tc_root(x, W_r, b_l):
    return pl.pallas_call(
        _tc_root_kernel,
        grid=(_N // _BLK,),
        in_specs=[
            pl.BlockSpec((_BLK, _D), lambda i: (i, 0)),
            pl.BlockSpec((_D, _D), lambda i: (0, 0)),
            pl.BlockSpec((1, _D), lambda i: (0, 0)),
        ],
        out_specs=pl.BlockSpec((_BLK, _D), lambda i: (i, 0)),
        out_shape=jax.ShapeDtypeStruct((_N, _D), jnp.float32),
    )(x, W_r, b_l)


def _tc_combine_kernel(r_ref, alo_ref, ahi_ref, deg_ref, wl_ref, o_ref):
    agg = jnp.concatenate([alo_ref[...], ahi_ref[...]], axis=1)
    deg = jnp.maximum(deg_ref[...], 1.0)
    mean = agg / deg
    h = lax.dot_general(mean, wl_ref[...], (((1,), (1,)), ((), ())),
                        preferred_element_type=jnp.float32)
    h = h + r_ref[...]
    ss = jnp.sum(h * h, axis=1, keepdims=True)
    o_ref[...] = h / jnp.maximum(jnp.sqrt(ss), 1e-12)


def _tc_combine(r, alo, ahi, deg, W_l):
    hspec = pl.BlockSpec((_BLK, _H), lambda i: (i, 0))
    return pl.pallas_call(
        _tc_combine_kernel,
        grid=(_N // _BLK,),
        in_specs=[
            pl.BlockSpec((_BLK, _D), lambda i: (i, 0)),
            hspec, hspec,
            pl.BlockSpec((_BLK, 1), lambda i: (i, 0)),
            pl.BlockSpec((_D, _D), lambda i: (0, 0)),
        ],
        out_specs=pl.BlockSpec((_BLK, _D), lambda i: (i, 0)),
        out_shape=jax.ShapeDtypeStruct((_N, _D), jnp.float32),
    )(r, alo, ahi, deg, W_l)


@jax.jit
def kernel(x, edge_index, W_l, W_r, b_l):
    src = edge_index[0].astype(jnp.int32).reshape(16, _JCH, _CHUNK)
    dst = edge_index[1].astype(jnp.int32).reshape(16, _JCH, _CHUNK)
    x_lo = x[:, :_H]
    x_hi = x[:, _H:]
    zrow = jnp.zeros((_RPT, _H), jnp.float32)
    zdeg = jnp.zeros((_NPAD,), jnp.float32)
    onesh = jnp.ones((_CHUNK,), jnp.float32)
    alo, ahi, deg = _sc_agg(x_lo, x_hi, src, dst, zrow, zdeg, onesh)
    r = _tc_root(x, W_r, b_l.reshape(1, _D))
    return _tc_combine(r, alo, ahi, deg.reshape(_NPAD, 1), W_l)


# speedup vs baseline: 1.0024x; 1.0024x over previous
import jax
import jax.numpy as jnp
from jax import lax
from jax.experimental import pallas as pl
from jax.experimental.pallas import tpu as pltpu
from jax.experimental.pallas import tpu_sc as plsc

_N = 10000
_E = 160000
_D = 256
_H = 128
_CHUNK = 100
_JCH = 100
_RPT = 640
_NPAD = 10240
_WB = 80


def _sc_agg_kernel(x_lo, x_hi, src3d, dst3d, zrow, zdeg, onesh,
                   agg_lo, agg_hi, deg_out,
                   idx_src, idx_dst, rows, ones_v, dbuf,
                   sga, sgb, ssa, ssb, sd,
                   acc_sh, deg_sh):
    c = lax.axis_index("c")
    s = lax.axis_index("s")

    pltpu.sync_copy(src3d.at[s], idx_src)
    pltpu.sync_copy(dst3d.at[s], idx_dst)

    def run(x_tab, agg_out, with_deg):
        pltpu.sync_copy(zrow, acc_sh.at[pl.ds(s * _RPT, _RPT)])
        if with_deg:
            pltpu.sync_copy(zdeg.at[pl.ds(s * _RPT, _RPT)],
                            deg_sh.at[pl.ds(s * _RPT, _RPT)])
            pltpu.sync_copy(onesh, ones_v)
        plsc.subcore_barrier()

        def gather(j, buf, sem):
            pltpu.async_copy(x_tab.at[idx_src.at[j]], rows.at[buf], sem)

        def gather_wait(j, buf, sem):
            pltpu.make_async_copy(x_tab.at[idx_src.at[j]], rows.at[buf],
                                  sem).wait()

        def scatter_start(j, buf, sem):
            ds = [pltpu.async_copy(rows.at[buf], acc_sh.at[idx_dst.at[j]],
                                   sem, add=True)]
            if with_deg:
                ds.append(pltpu.async_copy(ones_v, deg_sh.at[idx_dst.at[j]],
                                           sd, add=True))
            return ds

        gather(0, 0, sga)
        gather(1, 1, sgb)

        def body(i, carry):
            a = 2 * i
            gather_wait(a, 0, sga)
            da = scatter_start(a, 0, ssa)
            gather_wait(a + 1, 1, sgb)
            db = scatter_start(a + 1, 1, ssb)
            for d in da:
                d.wait()

            @pl.when(a + 2 < _JCH)
            def _():
                gather(a + 2, 0, sga)
            for d in db:
                d.wait()

            @pl.when(a + 3 < _JCH)
            def _():
                gather(a + 3, 1, sgb)
            return carry

        lax.fori_loop(0, _JCH // 2, body, 0)
        if _JCH % 2 == 1:
            gather_wait(_JCH - 1, 0, sga)
            for d in scatter_start(_JCH - 1, 0, ssa):
                d.wait()

        plsc.subcore_barrier()

        pltpu.sync_copy(acc_sh.at[pl.ds(s * _RPT, _WB)],
                        rows.at[0, pl.ds(0, _WB)])
        for k in range(8):
            b = k % 2
            if k < 7:
                r1 = s * _RPT + (k + 1) * _WB
                pltpu.async_copy(acc_sh.at[pl.ds(r1, _WB)],
                                 rows.at[1 - b, pl.ds(0, _WB)], sga)
            r0 = s * _RPT + k * _WB
            pltpu.sync_copy(rows.at[b, pl.ds(0, _WB)],
                            agg_out.at[pl.ds(r0, _WB)])
            if k < 7:
                r1 = s * _RPT + (k + 1) * _WB
                pltpu.make_async_copy(acc_sh.at[pl.ds(r1, _WB)],
                                      rows.at[1 - b, pl.ds(0, _WB)],
                                      sga).wait()
        if with_deg:
            pltpu.sync_copy(deg_sh.at[pl.ds(s * _RPT, _RPT)], dbuf)
            pltpu.sync_copy(dbuf, deg_out.at[pl.ds(s * _RPT, _RPT)])

    @pl.when(c == 0)
    def _():
        run(x_lo, agg_lo, True)

    @pl.when(c == 1)
    def _():
        run(x_hi, agg_hi, False)


_sc_agg = pl.kernel(
    _sc_agg_kernel,
    out_type=[
        jax.ShapeDtypeStruct((_NPAD, _H), jnp.float32),
        jax.ShapeDtypeStruct((_NPAD, _H), jnp.float32),
        jax.ShapeDtypeStruct((_NPAD,), jnp.float32),
    ],
    mesh=plsc.VectorSubcoreMesh(core_axis_name="c", subcore_axis_name="s"),
    compiler_params=pltpu.CompilerParams(use_tc_tiling_on_sc=False),
    scratch_types=[
        pltpu.VMEM((_JCH, _CHUNK), jnp.int32),
        pltpu.VMEM((_JCH, _CHUNK), jnp.int32),
        pltpu.VMEM((2, _CHUNK, _H), jnp.float32),
        pltpu.VMEM((_CHUNK,), jnp.float32),
        pltpu.VMEM((_RPT,), jnp.float32),
        pltpu.SemaphoreType.DMA,
        pltpu.SemaphoreType.DMA,
        pltpu.SemaphoreType.DMA,
        pltpu.SemaphoreType.DMA,
        pltpu.SemaphoreType.DMA,
        pltpu.VMEM_SHARED((_NPAD, _H), jnp.float32),
        pltpu.VMEM_SHARED((_NPAD,), jnp.float32),
    ],
)


_BLK = 2000


def _tc_root_kernel(x_ref, wr_ref, b_ref, o_ref):
    o_ref[...] = lax.dot_general(
        x_ref[...], wr_ref[...], (((1,), (1,)), ((), ())),
        preferred_element_type=jnp.float32) + b_ref[...]


def _tc_root(x, W_r, b_l):
    return pl.pallas_call(
        _tc_root_kernel,
        grid=(_N // _BLK,),
        in_specs=[
            pl.BlockSpec((_BLK, _D), lambda i: (i, 0)),
            pl.BlockSpec((_D, _D), lambda i: (0, 0)),
            pl.BlockSpec((1, _D), lambda i: (0, 0)),
        ],
        out_specs=pl.BlockSpec((_BLK, _D), lambda i: (i, 0)),
        out_shape=jax.ShapeDtypeStruct((_N, _D), jnp.float32),
    )(x, W_r, b_l)


def _tc_combine_kernel(r_ref, alo_ref, ahi_ref, deg_ref, wl_ref, o_ref):
    agg = jnp.concatenate([alo_ref[...], ahi_ref[...]], axis=1)
    deg = jnp.maximum(deg_ref[...], 1.0)
    mean = agg / deg
    h = lax.dot_general(mean, wl_ref[...], (((1,), (1,)), ((), ())),
                        preferred_element_type=jnp.float32)
    h = h + r_ref[...]
    ss = jnp.sum(h * h, axis=1, keepdims=True)
    o_ref[...] = h / jnp.maximum(jnp.sqrt(ss), 1e-12)


def _tc_combine(r, alo, ahi, deg, W_l):
    hspec = pl.BlockSpec((_BLK, _H), lambda i: (i, 0))
    return pl.pallas_call(
        _tc_combine_kernel,
        grid=(_N // _BLK,),
        in_specs=[
            pl.BlockSpec((_BLK, _D), lambda i: (i, 0)),
            hspec, hspec,
            pl.BlockSpec((_BLK, 1), lambda i: (i, 0)),
            pl.BlockSpec((_D, _D), lambda i: (0, 0)),
        ],
        out_specs=pl.BlockSpec((_BLK, _D), lambda i: (i, 0)),
        out_shape=jax.ShapeDtypeStruct((_N, _D), jnp.float32),
    )(r, alo, ahi, deg, W_l)


@jax.jit
def kernel(x, edge_index, W_l, W_r, b_l):
    src = edge_index[0].astype(jnp.int32).reshape(16, _JCH, _CHUNK)
    dst = edge_index[1].astype(jnp.int32).reshape(16, _JCH, _CHUNK)
    x_lo = x[:, :_H]
    x_hi = x[:, _H:]
    zrow = jnp.zeros((_RPT, _H), jnp.float32)
    zdeg = jnp.zeros((_NPAD,), jnp.float32)
    onesh = jnp.ones((_CHUNK,), jnp.float32)
    alo, ahi, deg = _sc_agg(x_lo, x_hi, src, dst, zrow, zdeg, onesh)
    r = _tc_root(x, W_r, b_l.reshape(1, _D))
    return _tc_combine(r, alo, ahi, deg.reshape(_NPAD, 1), W_l)
